# baseline (speedup 1.0000x reference)
import jax
import jax.numpy as jnp
from jax import lax
from jax.experimental import pallas as pl
from jax.experimental.pallas import tpu as pltpu
from jax.experimental.pallas import tpu_sc as plsc

_B = 16384
_D = 64
_NC = 2
_NS = 16
_L = 16
_NW = _NC * _NS
_BPW = _B // _NW
_CH = 128
_NCH = _BPW // _CH
_NE2 = 500000
_NR2 = 500


_NWIN = 1000000 // 128
_TAIL = 1000000 - _NWIN * 128
_WPT = (_NWIN + _NW) // _NW


def _tbody(ent_t_hbm, out_hbm, win0, win1, ob0, ob1, semi0, semi1, semo0, semo1):
    wid = lax.axis_index("s") * _NC + lax.axis_index("c")
    w0 = wid * _WPT
    wlim = jnp.minimum(w0 + _WPT, _NWIN)
    iota = lax.iota(jnp.int32, _L)
    colv = (iota & 1) << 6
    rhalf = iota >> 1

    wins, obs = (win0, win1), (ob0, ob1)
    semis, semos = (semi0, semi1), (semo0, semo1)

    def win_src(w):
        return ent_t_hbm.at[:, pl.ds(w * 128, 128)]

    def out_dst(w):
        return out_hbm.at[pl.ds(w * 64, 64), :]

    def issue_in(w, buf, sem):
        @pl.when(w < wlim)
        def _():
            pltpu.make_async_copy(win_src(w), buf, sem).start()

    issue_in(w0, win0, semi0)
    issue_in(w0 + 1, win1, semi1)

    @pl.loop(0, _WPT, step=2)
    def _(k):
        for s in range(2):
            w = w0 + k + s
            buf, ob = wins[s], obs[s]
            semi, semo = semis[s], semos[s]

            @pl.when((k + s >= 2) & (w - 2 < wlim))
            def _():
                pltpu.make_async_copy(ob, out_dst(w0), semo).wait()

            @pl.when(w < wlim)
            def _():
                pltpu.make_async_copy(win_src(w), buf, semi).wait()

                @pl.loop(0, 8)
                def _(v, buf=buf, ob=ob):
                    evec = iota + (v << 4)
                    rows = rhalf + (v << 3)
                    for c in range(64):
                        cvec = (iota + c) & 63
                        val = plsc.load_gather(buf, [cvec, evec])
                        plsc.store_scatter(ob, [rows, colv + cvec], val)

                pltpu.make_async_copy(ob, out_dst(w), semo).start()
                issue_in(w + 2, buf, semi)

    @pl.when(w0 + _WPT - 1 < wlim)
    def _():
        pltpu.make_async_copy(ob0, out_dst(w0), semo0).wait()

    @pl.when(wid == _NW - 1)
    def _():
        for c in range(_D):
            pltpu.sync_copy(ent_t_hbm.at[c, pl.ds(_NWIN * 128, _TAIL)],
                            win0.at[c, pl.ds(0, _TAIL)])
        @pl.loop(0, _TAIL // 16)
        def _(v):
            evec = iota + (v << 4)
            rows = rhalf + (v << 3)
            for c in range(64):
                cvec = (iota + c) & 63
                val = plsc.load_gather(win0, [cvec, evec])
                plsc.store_scatter(ob0, [rows, colv + cvec], val)
        pltpu.sync_copy(ob0.at[pl.ds(0, _TAIL // 2), :],
                        out_hbm.at[pl.ds(_NWIN * 64, _TAIL // 2), :])


def _sqrt16(x):
    i = plsc.bitcast(x, jnp.int32)
    i = (i >> 1) + jnp.int32(0x1FBD1DF6)
    y = plsc.bitcast(i, jnp.float32)
    for _ in range(3):
        y = 0.5 * (y + x / y)
    return y


def _body(ent_hbm, rel_hbm, hidx_hbm, tidx_hbm, lidx_hbm, out_hbm,
          hidx_v, tidx_v, lidx_v, hp_v, tp_v, lp_v,
          gh0, gt0, gr0, gh1, gt1, gr1, out_v,
          sem0, sem1):
    wid = lax.axis_index("s") * _NC + lax.axis_index("c")
    base = wid * _BPW

    for c in range(_NCH):
        pltpu.sync_copy(hidx_hbm.at[wid * _NCH + c], hidx_v.at[c])
        pltpu.sync_copy(tidx_hbm.at[wid * _NCH + c], tidx_v.at[c])
        pltpu.sync_copy(lidx_hbm.at[wid * _NCH + c], lidx_v.at[c])
    for c in range(_NCH):
        for o in range(_CH // _L):
            sl = pl.ds(o * _L, _L)
            hp_v[c, sl] = hidx_v[c, sl] >> 1
            tp_v[c, sl] = tidx_v[c, sl] >> 1
            lp_v[c, sl] = lidx_v[c, sl] >> 1

    ghs, gts, grs = (gh0, gh1), (gt0, gt1), (gr0, gr1)
    sems = (sem0, sem1)

    ent_copies = {}
    rel_copies = {}
    for b in range(2):
        ent_copies[b] = (
            pltpu.async_copy(ent_hbm.at[hp_v.at[b]], ghs[b], sems[b]),
            pltpu.async_copy(ent_hbm.at[tp_v.at[b]], gts[b], sems[b]),
        )
        rel_copies[b] = pltpu.async_copy(
            rel_hbm.at[lp_v.at[b]], grs[b], sems[b])

    lane = lax.iota(jnp.int32, _L)

    for b in range(_NCH):
        buf = b % 2
        for cp in ent_copies.pop(b):
            cp.wait()
        rel_copies.pop(b).wait()
        gh, gt, gr = ghs[buf], gts[buf], grs[buf]

        @pl.loop(0, _CH // _L)
        def _(g2, b=b, gh=gh, gt=gt, gr=gr):
            sl = pl.ds(g2 * _L, _L)
            qh = (hidx_v[b, sl] & 1) << 6
            qt = (tidx_v[b, sl] & 1) << 6
            qr = (lidx_v[b, sl] & 1) << 6
            rows = lane + g2 * _L
            acc = jnp.zeros((_L,), jnp.float32)
            for d in range(_D):
                off = (lane + d) & (_D - 1)
                vh = plsc.load_gather(gh, [rows, qh + off])
                vt = plsc.load_gather(gt, [rows, qt + off])
                vr = plsc.load_gather(gr, [rows, qr + off])
                s = vh + vr - vt
                acc = acc + s * s
            out_v[pl.ds(b * _CH + g2 * _L, _L)] = _sqrt16(acc)

        nxt = b + 2
        if nxt < _NCH:
            ent_copies[nxt] = (
                pltpu.async_copy(ent_hbm.at[hp_v.at[nxt]], gh, sems[buf]),
                pltpu.async_copy(ent_hbm.at[tp_v.at[nxt]], gt, sems[buf]),
            )
            rel_copies[nxt] = pltpu.async_copy(
                rel_hbm.at[lp_v.at[nxt]], gr, sems[buf])

    pltpu.sync_copy(out_v, out_hbm.at[pl.ds(base, _BPW)])


@jax.jit
def _transe_sc(head, tail, label, entity_emb, relation_emb):
    mesh_t = plsc.VectorSubcoreMesh(core_axis_name="c", subcore_axis_name="s")
    cp_t = pltpu.CompilerParams(
        needs_layout_passes=False, use_tc_tiling_on_sc=True
    )
    kt = pl.kernel(
        _tbody,
        out_type=jax.ShapeDtypeStruct((_NE2, 2 * _D), jnp.float32),
        mesh=mesh_t,
        scratch_types=[
            pltpu.VMEM((_D, 128), jnp.float32),
            pltpu.VMEM((_D, 128), jnp.float32),
            pltpu.VMEM((_D, 2 * _D), jnp.float32),
            pltpu.VMEM((_D, 2 * _D), jnp.float32),
            pltpu.SemaphoreType.DMA,
            pltpu.SemaphoreType.DMA,
            pltpu.SemaphoreType.DMA,
            pltpu.SemaphoreType.DMA,
        ],
        compiler_params=cp_t,
    )
    ent2 = kt(entity_emb.T)
    rel2 = relation_emb.reshape(_NR2, 2 * _D)
    hidx = head.astype(jnp.int32).reshape(_NW * _NCH, _CH)
    tidx = tail.astype(jnp.int32).reshape(_NW * _NCH, _CH)
    lidx = label.astype(jnp.int32).reshape(_NW * _NCH, _CH)
    mesh = plsc.VectorSubcoreMesh(core_axis_name="c", subcore_axis_name="s")
    cp = pltpu.CompilerParams(
        needs_layout_passes=False, use_tc_tiling_on_sc=True
    )
    k = pl.kernel(
        _body,
        out_type=jax.ShapeDtypeStruct((_B,), jnp.float32),
        mesh=mesh,
        scratch_types=[
            pltpu.VMEM((_NCH, _CH), jnp.int32),
            pltpu.VMEM((_NCH, _CH), jnp.int32),
            pltpu.VMEM((_NCH, _CH), jnp.int32),
            pltpu.VMEM((_NCH, _CH), jnp.int32),
            pltpu.VMEM((_NCH, _CH), jnp.int32),
            pltpu.VMEM((_NCH, _CH), jnp.int32),
            pltpu.VMEM((_CH, 2 * _D), jnp.float32),
            pltpu.VMEM((_CH, 2 * _D), jnp.float32),
            pltpu.VMEM((_CH, 2 * _D), jnp.float32),
            pltpu.VMEM((_CH, 2 * _D), jnp.float32),
            pltpu.VMEM((_CH, 2 * _D), jnp.float32),
            pltpu.VMEM((_CH, 2 * _D), jnp.float32),
            pltpu.VMEM((_BPW,), jnp.float32),
            pltpu.SemaphoreType.DMA,
            pltpu.SemaphoreType.DMA,
        ],
        compiler_params=cp,
    )
    return k(ent2, rel2, hidx, tidx, lidx)


def kernel(head, tail, label, entity_emb, relation_emb):
    return _transe_sc(head, tail, label, entity_emb, relation_emb)


# speedup vs baseline: 1.2746x; 1.2746x over previous
import jax
import jax.numpy as jnp
from jax import lax
from jax.experimental import pallas as pl
from jax.experimental.pallas import tpu as pltpu
from jax.experimental.pallas import tpu_sc as plsc

_B = 16384
_D = 64
_NC = 2
_NS = 16
_L = 16
_NW = _NC * _NS
_BPW = _B // _NW
_CH = 128
_NCH = _BPW // _CH
_NE2 = 500000
_NR2 = 500


_NWIN = 1000000 // 128
_TAIL = 1000000 - _NWIN * 128
_WPT = (_NWIN + _NW) // _NW


def _tbody(ent_t_hbm, out_hbm, win0, win1, ob0, ob1, semi0, semi1, semo0, semo1):
    wid = lax.axis_index("s") * _NC + lax.axis_index("c")
    w0 = wid * _WPT
    wlim = jnp.minimum(w0 + _WPT, _NWIN)
    iota = lax.iota(jnp.int32, _L)
    colv = (iota & 1) << 6
    rhalf = iota >> 1

    wins, obs = (win0, win1), (ob0, ob1)
    semis, semos = (semi0, semi1), (semo0, semo1)

    def win_src(w):
        return ent_t_hbm.at[:, pl.ds(w * 128, 128)]

    def out_dst(w):
        return out_hbm.at[pl.ds(w * 64, 64), :]

    def issue_in(w, buf, sem):
        @pl.when(w < wlim)
        def _():
            pltpu.make_async_copy(win_src(w), buf, sem).start()

    issue_in(w0, win0, semi0)
    issue_in(w0 + 1, win1, semi1)

    @pl.loop(0, _WPT, step=2)
    def _(k):
        for s in range(2):
            w = w0 + k + s
            buf, ob = wins[s], obs[s]
            semi, semo = semis[s], semos[s]

            @pl.when((k + s >= 2) & (w - 2 < wlim))
            def _():
                pltpu.make_async_copy(ob, out_dst(w0), semo).wait()

            @pl.when(w < wlim)
            def _():
                pltpu.make_async_copy(win_src(w), buf, semi).wait()

                @plsc.parallel_loop(0, 8, unroll=2)
                def _(v, buf=buf, ob=ob):
                    evec = iota + (v << 4)
                    rows = rhalf + (v << 3)
                    for c in range(64):
                        cvec = (iota + c) & 63
                        val = plsc.load_gather(buf, [cvec, evec])
                        plsc.store_scatter(ob, [rows, colv + cvec], val)

                pltpu.make_async_copy(ob, out_dst(w), semo).start()
                issue_in(w + 2, buf, semi)

    @pl.when(w0 + _WPT - 1 < wlim)
    def _():
        pltpu.make_async_copy(ob0, out_dst(w0), semo0).wait()

    @pl.when(wid == _NW - 1)
    def _():
        for c in range(_D):
            pltpu.sync_copy(ent_t_hbm.at[c, pl.ds(_NWIN * 128, _TAIL)],
                            win0.at[c, pl.ds(0, _TAIL)])
        @plsc.parallel_loop(0, _TAIL // 16)
        def _(v):
            evec = iota + (v << 4)
            rows = rhalf + (v << 3)
            for c in range(64):
                cvec = (iota + c) & 63
                val = plsc.load_gather(win0, [cvec, evec])
                plsc.store_scatter(ob0, [rows, colv + cvec], val)
        pltpu.sync_copy(ob0.at[pl.ds(0, _TAIL // 2), :],
                        out_hbm.at[pl.ds(_NWIN * 64, _TAIL // 2), :])


def _sqrt16(x):
    i = plsc.bitcast(x, jnp.int32)
    i = (i >> 1) + jnp.int32(0x1FBD1DF6)
    y = plsc.bitcast(i, jnp.float32)
    for _ in range(3):
        y = 0.5 * (y + x / y)
    return y


def _body(ent_hbm, rel_hbm, hidx_hbm, tidx_hbm, lidx_hbm, out_hbm,
          hidx_v, tidx_v, lidx_v, hp_v, tp_v, lp_v,
          gh0, gt0, gr0, gh1, gt1, gr1, out_v,
          sem0, sem1):
    wid = lax.axis_index("s") * _NC + lax.axis_index("c")
    base = wid * _BPW

    for c in range(_NCH):
        pltpu.sync_copy(hidx_hbm.at[wid * _NCH + c], hidx_v.at[c])
        pltpu.sync_copy(tidx_hbm.at[wid * _NCH + c], tidx_v.at[c])
        pltpu.sync_copy(lidx_hbm.at[wid * _NCH + c], lidx_v.at[c])
    for c in range(_NCH):
        for o in range(_CH // _L):
            sl = pl.ds(o * _L, _L)
            hp_v[c, sl] = hidx_v[c, sl] >> 1
            tp_v[c, sl] = tidx_v[c, sl] >> 1
            lp_v[c, sl] = lidx_v[c, sl] >> 1

    ghs, gts, grs = (gh0, gh1), (gt0, gt1), (gr0, gr1)
    sems = (sem0, sem1)

    ent_copies = {}
    rel_copies = {}
    for b in range(2):
        ent_copies[b] = (
            pltpu.async_copy(ent_hbm.at[hp_v.at[b]], ghs[b], sems[b]),
            pltpu.async_copy(ent_hbm.at[tp_v.at[b]], gts[b], sems[b]),
        )
        rel_copies[b] = pltpu.async_copy(
            rel_hbm.at[lp_v.at[b]], grs[b], sems[b])

    lane = lax.iota(jnp.int32, _L)

    for b in range(_NCH):
        buf = b % 2
        for cp in ent_copies.pop(b):
            cp.wait()
        rel_copies.pop(b).wait()
        gh, gt, gr = ghs[buf], gts[buf], grs[buf]

        @pl.loop(0, _CH // _L)
        def _(g2, b=b, gh=gh, gt=gt, gr=gr):
            sl = pl.ds(g2 * _L, _L)
            qh = (hidx_v[b, sl] & 1) << 6
            qt = (tidx_v[b, sl] & 1) << 6
            qr = (lidx_v[b, sl] & 1) << 6
            rows = lane + g2 * _L
            acc = jnp.zeros((_L,), jnp.float32)
            for d in range(_D):
                off = (lane + d) & (_D - 1)
                vh = plsc.load_gather(gh, [rows, qh + off])
                vt = plsc.load_gather(gt, [rows, qt + off])
                vr = plsc.load_gather(gr, [rows, qr + off])
                s = vh + vr - vt
                acc = acc + s * s
            out_v[pl.ds(b * _CH + g2 * _L, _L)] = _sqrt16(acc)

        nxt = b + 2
        if nxt < _NCH:
            ent_copies[nxt] = (
                pltpu.async_copy(ent_hbm.at[hp_v.at[nxt]], gh, sems[buf]),
                pltpu.async_copy(ent_hbm.at[tp_v.at[nxt]], gt, sems[buf]),
            )
            rel_copies[nxt] = pltpu.async_copy(
                rel_hbm.at[lp_v.at[nxt]], gr, sems[buf])

    pltpu.sync_copy(out_v, out_hbm.at[pl.ds(base, _BPW)])


@jax.jit
def _transe_sc(head, tail, label, entity_emb, relation_emb):
    mesh_t = plsc.VectorSubcoreMesh(core_axis_name="c", subcore_axis_name="s")
    cp_t = pltpu.CompilerParams(
        needs_layout_passes=False, use_tc_tiling_on_sc=True
    )
    kt = pl.kernel(
        _tbody,
        out_type=jax.ShapeDtypeStruct((_NE2, 2 * _D), jnp.float32),
        mesh=mesh_t,
        scratch_types=[
            pltpu.VMEM((_D, 128), jnp.float32),
            pltpu.VMEM((_D, 128), jnp.float32),
            pltpu.VMEM((_D, 2 * _D), jnp.float32),
            pltpu.VMEM((_D, 2 * _D), jnp.float32),
            pltpu.SemaphoreType.DMA,
            pltpu.SemaphoreType.DMA,
            pltpu.SemaphoreType.DMA,
            pltpu.SemaphoreType.DMA,
        ],
        compiler_params=cp_t,
    )
    ent2 = kt(entity_emb.T)
    rel2 = relation_emb.reshape(_NR2, 2 * _D)
    hidx = head.astype(jnp.int32).reshape(_NW * _NCH, _CH)
    tidx = tail.astype(jnp.int32).reshape(_NW * _NCH, _CH)
    lidx = label.astype(jnp.int32).reshape(_NW * _NCH, _CH)
    mesh = plsc.VectorSubcoreMesh(core_axis_name="c", subcore_axis_name="s")
    cp = pltpu.CompilerParams(
        needs_layout_passes=False, use_tc_tiling_on_sc=True
    )
    k = pl.kernel(
        _body,
        out_type=jax.ShapeDtypeStruct((_B,), jnp.float32),
        mesh=mesh,
        scratch_types=[
            pltpu.VMEM((_NCH, _CH), jnp.int32),
            pltpu.VMEM((_NCH, _CH), jnp.int32),
            pltpu.VMEM((_NCH, _CH), jnp.int32),
            pltpu.VMEM((_NCH, _CH), jnp.int32),
            pltpu.VMEM((_NCH, _CH), jnp.int32),
            pltpu.VMEM((_NCH, _CH), jnp.int32),
            pltpu.VMEM((_CH, 2 * _D), jnp.float32),
            pltpu.VMEM((_CH, 2 * _D), jnp.float32),
            pltpu.VMEM((_CH, 2 * _D), jnp.float32),
            pltpu.VMEM((_CH, 2 * _D), jnp.float32),
            pltpu.VMEM((_CH, 2 * _D), jnp.float32),
            pltpu.VMEM((_CH, 2 * _D), jnp.float32),
            pltpu.VMEM((_BPW,), jnp.float32),
            pltpu.SemaphoreType.DMA,
            pltpu.SemaphoreType.DMA,
        ],
        compiler_params=cp,
    )
    return k(ent2, rel2, hidx, tidx, lidx)


def kernel(head, tail, label, entity_emb, relation_emb):
    return _transe_sc(head, tail, label, entity_emb, relation_emb)


# speedup vs baseline: 2.5209x; 1.9778x over previous
import jax
import jax.numpy as jnp
from jax import lax
from jax.experimental import pallas as pl
from jax.experimental.pallas import tpu as pltpu
from jax.experimental.pallas import tpu_sc as plsc

_B = 16384
_D = 64
_NC = 2
_NS = 16
_L = 16
_NW = _NC * _NS
_BPW = _B // _NW
_CH = 128
_NCH = _BPW // _CH
_NE2 = 500000
_NR2 = 500


_NWIN = 1000000 // 128
_TAIL = 1000000 - _NWIN * 128
_WPT = (_NWIN + _NW) // _NW


def _tbody(ent_t_hbm, out_hbm, win0, win1, ob0, ob1, semi0, semi1, semo0, semo1):
    wid = lax.axis_index("s") * _NC + lax.axis_index("c")
    w0 = wid * _WPT
    wlim = jnp.minimum(w0 + _WPT, _NWIN)
    iota = lax.iota(jnp.int32, _L)
    colv = (iota & 1) << 6
    rhalf = iota >> 1

    wins, obs = (win0, win1), (ob0, ob1)
    semis, semos = (semi0, semi1), (semo0, semo1)

    def win_src(w):
        return ent_t_hbm.at[:, pl.ds(w * 128, 128)]

    def out_dst(w):
        return out_hbm.at[pl.ds(w * 64, 64), :]

    def issue_in(w, buf, sem):
        @pl.when(w < wlim)
        def _():
            pltpu.make_async_copy(win_src(w), buf, sem).start()

    issue_in(w0, win0, semi0)
    issue_in(w0 + 1, win1, semi1)

    @pl.loop(0, _WPT, step=2)
    def _(k):
        for s in range(2):
            w = w0 + k + s
            buf, ob = wins[s], obs[s]
            semi, semo = semis[s], semos[s]

            @pl.when((k + s >= 2) & (w - 2 < wlim))
            def _():
                pltpu.make_async_copy(ob, out_dst(w0), semo).wait()

            @pl.when(w < wlim)
            def _():
                pltpu.make_async_copy(win_src(w), buf, semi).wait()

                @plsc.parallel_loop(0, 8, unroll=4)
                def _(v, buf=buf, ob=ob):
                    evec = iota + (v << 4)
                    rows = rhalf + (v << 3)
                    for c in range(64):
                        cvec = (iota + c) & 63
                        val = plsc.load_gather(buf, [cvec, evec])
                        plsc.store_scatter(ob, [rows, colv + cvec], val)

                pltpu.make_async_copy(ob, out_dst(w), semo).start()
                issue_in(w + 2, buf, semi)

    @pl.when(w0 + _WPT - 1 < wlim)
    def _():
        pltpu.make_async_copy(ob0, out_dst(w0), semo0).wait()

    @pl.when(wid == _NW - 1)
    def _():
        for c in range(_D):
            pltpu.sync_copy(ent_t_hbm.at[c, pl.ds(_NWIN * 128, _TAIL)],
                            win0.at[c, pl.ds(0, _TAIL)])
        @plsc.parallel_loop(0, _TAIL // 16)
        def _(v):
            evec = iota + (v << 4)
            rows = rhalf + (v << 3)
            for c in range(64):
                cvec = (iota + c) & 63
                val = plsc.load_gather(win0, [cvec, evec])
                plsc.store_scatter(ob0, [rows, colv + cvec], val)
        pltpu.sync_copy(ob0.at[pl.ds(0, _TAIL // 2), :],
                        out_hbm.at[pl.ds(_NWIN * 64, _TAIL // 2), :])


def _sqrt16(x):
    i = plsc.bitcast(x, jnp.int32)
    i = (i >> 1) + jnp.int32(0x1FBD1DF6)
    y = plsc.bitcast(i, jnp.float32)
    for _ in range(3):
        y = 0.5 * (y + x / y)
    return y


def _body(ent_hbm, rel_hbm, hidx_hbm, tidx_hbm, lidx_hbm, out_hbm,
          hidx_v, tidx_v, lidx_v, hp_v, tp_v, lp_v,
          gh0, gt0, gr0, gh1, gt1, gr1, out_v,
          sem0, sem1):
    wid = lax.axis_index("s") * _NC + lax.axis_index("c")
    base = wid * _BPW

    for c in range(_NCH):
        pltpu.sync_copy(hidx_hbm.at[wid * _NCH + c], hidx_v.at[c])
        pltpu.sync_copy(tidx_hbm.at[wid * _NCH + c], tidx_v.at[c])
        pltpu.sync_copy(lidx_hbm.at[wid * _NCH + c], lidx_v.at[c])
    for c in range(_NCH):
        for o in range(_CH // _L):
            sl = pl.ds(o * _L, _L)
            hp_v[c, sl] = hidx_v[c, sl] >> 1
            tp_v[c, sl] = tidx_v[c, sl] >> 1
            lp_v[c, sl] = lidx_v[c, sl] >> 1

    ghs, gts, grs = (gh0, gh1), (gt0, gt1), (gr0, gr1)
    sems = (sem0, sem1)

    ent_copies = {}
    rel_copies = {}
    for b in range(2):
        ent_copies[b] = (
            pltpu.async_copy(ent_hbm.at[hp_v.at[b]], ghs[b], sems[b]),
            pltpu.async_copy(ent_hbm.at[tp_v.at[b]], gts[b], sems[b]),
        )
        rel_copies[b] = pltpu.async_copy(
            rel_hbm.at[lp_v.at[b]], grs[b], sems[b])

    lane = lax.iota(jnp.int32, _L)

    for b in range(_NCH):
        buf = b % 2
        for cp in ent_copies.pop(b):
            cp.wait()
        rel_copies.pop(b).wait()
        gh, gt, gr = ghs[buf], gts[buf], grs[buf]

        @pl.loop(0, _CH // _L)
        def _(g2, b=b, gh=gh, gt=gt, gr=gr):
            sl = pl.ds(g2 * _L, _L)
            qh = (hidx_v[b, sl] & 1) << 6
            qt = (tidx_v[b, sl] & 1) << 6
            qr = (lidx_v[b, sl] & 1) << 6
            rows = lane + g2 * _L
            acc = jnp.zeros((_L,), jnp.float32)
            for d in range(_D):
                off = (lane + d) & (_D - 1)
                vh = plsc.load_gather(gh, [rows, qh + off])
                vt = plsc.load_gather(gt, [rows, qt + off])
                vr = plsc.load_gather(gr, [rows, qr + off])
                s = vh + vr - vt
                acc = acc + s * s
            out_v[pl.ds(b * _CH + g2 * _L, _L)] = _sqrt16(acc)

        nxt = b + 2
        if nxt < _NCH:
            ent_copies[nxt] = (
                pltpu.async_copy(ent_hbm.at[hp_v.at[nxt]], gh, sems[buf]),
                pltpu.async_copy(ent_hbm.at[tp_v.at[nxt]], gt, sems[buf]),
            )
            rel_copies[nxt] = pltpu.async_copy(
                rel_hbm.at[lp_v.at[nxt]], gr, sems[buf])

    pltpu.sync_copy(out_v, out_hbm.at[pl.ds(base, _BPW)])


@jax.jit
def _transe_sc(head, tail, label, entity_emb, relation_emb):
    mesh_t = plsc.VectorSubcoreMesh(core_axis_name="c", subcore_axis_name="s")
    cp_t = pltpu.CompilerParams(
        needs_layout_passes=False, use_tc_tiling_on_sc=True
    )
    kt = pl.kernel(
        _tbody,
        out_type=jax.ShapeDtypeStruct((_NE2, 2 * _D), jnp.float32),
        mesh=mesh_t,
        scratch_types=[
            pltpu.VMEM((_D, 128), jnp.float32),
            pltpu.VMEM((_D, 128), jnp.float32),
            pltpu.VMEM((_D, 2 * _D), jnp.float32),
            pltpu.VMEM((_D, 2 * _D), jnp.float32),
            pltpu.SemaphoreType.DMA,
            pltpu.SemaphoreType.DMA,
            pltpu.SemaphoreType.DMA,
            pltpu.SemaphoreType.DMA,
        ],
        compiler_params=cp_t,
    )
    ent2 = kt(entity_emb.T)
    rel2 = relation_emb.reshape(_NR2, 2 * _D)
    hidx = head.astype(jnp.int32).reshape(_NW * _NCH, _CH)
    tidx = tail.astype(jnp.int32).reshape(_NW * _NCH, _CH)
    lidx = label.astype(jnp.int32).reshape(_NW * _NCH, _CH)
    mesh = plsc.VectorSubcoreMesh(core_axis_name="c", subcore_axis_name="s")
    cp = pltpu.CompilerParams(
        needs_layout_passes=False, use_tc_tiling_on_sc=True
    )
    k = pl.kernel(
        _body,
        out_type=jax.ShapeDtypeStruct((_B,), jnp.float32),
        mesh=mesh,
        scratch_types=[
            pltpu.VMEM((_NCH, _CH), jnp.int32),
            pltpu.VMEM((_NCH, _CH), jnp.int32),
            pltpu.VMEM((_NCH, _CH), jnp.int32),
            pltpu.VMEM((_NCH, _CH), jnp.int32),
            pltpu.VMEM((_NCH, _CH), jnp.int32),
            pltpu.VMEM((_NCH, _CH), jnp.int32),
            pltpu.VMEM((_CH, 2 * _D), jnp.float32),
            pltpu.VMEM((_CH, 2 * _D), jnp.float32),
            pltpu.VMEM((_CH, 2 * _D), jnp.float32),
            pltpu.VMEM((_CH, 2 * _D), jnp.float32),
            pltpu.VMEM((_CH, 2 * _D), jnp.float32),
            pltpu.VMEM((_CH, 2 * _D), jnp.float32),
            pltpu.VMEM((_BPW,), jnp.float32),
            pltpu.SemaphoreType.DMA,
            pltpu.SemaphoreType.DMA,
        ],
        compiler_params=cp,
    )
    return k(ent2, rel2, hidx, tidx, lidx)


def kernel(head, tail, label, entity_emb, relation_emb):
    return _transe_sc(head, tail, label, entity_emb, relation_emb)
